# baseline (speedup 1.0000x reference)
import functools

import jax
import jax.numpy as jnp
from jax import lax
from jax.experimental import pallas as pl
from jax.experimental.pallas import tpu as pltpu
from jax.experimental.pallas import tpu_sc as plsc

N = 10000
E = 320000
H = 128
L = 3

NC = 2
NS = 16
NW = NC * NS
EPW = E // NW
CHUNK = 80
NCHUNK = EPW // CHUNK
NPAIR = NCHUNK // 2
NPAD = 10240
RPW = NPAD // NS
ZROWS = 128

ROWBLK = 2000

_sc_mesh = plsc.VectorSubcoreMesh(core_axis_name="c", subcore_axis_name="s")
_sc_params = pltpu.CompilerParams(needs_layout_passes=False)


@functools.partial(
    pl.kernel,
    out_type=jax.ShapeDtypeStruct((E,), jnp.float32),
    mesh=_sc_mesh,
    compiler_params=_sc_params,
    scratch_types=[
        pltpu.VMEM((N,), jnp.float32),
        pltpu.VMEM((N,), jnp.float32),
        pltpu.VMEM((EPW,), jnp.int32),
        pltpu.VMEM((EPW,), jnp.int32),
        pltpu.VMEM((EPW,), jnp.float32),
    ],
)
def _edge_logits(a_hbm, b_hbm, src_hbm, dst_hbm, z_hbm,
                 a_v, b_v, src_v, dst_v, z_v):
    wid = lax.axis_index("s") * NC + lax.axis_index("c")
    base = wid * EPW
    pltpu.sync_copy(a_hbm, a_v)
    pltpu.sync_copy(b_hbm, b_v)
    pltpu.sync_copy(src_hbm.at[pl.ds(base, EPW)], src_v)
    pltpu.sync_copy(dst_hbm.at[pl.ds(base, EPW)], dst_v)

    def body(k, carry):
        off = k * 16
        si = src_v[pl.ds(off, 16)]
        di = dst_v[pl.ds(off, 16)]
        a = plsc.load_gather(a_v, [si])
        b = plsc.load_gather(b_v, [di])
        z = a + b
        z_v[pl.ds(off, 16)] = jnp.where(z >= 0.0, z, 0.2 * z)
        return carry

    lax.fori_loop(0, EPW // 16, body, 0)
    pltpu.sync_copy(z_v, z_hbm.at[pl.ds(base, EPW)])


@functools.partial(
    pl.kernel,
    out_type=jax.ShapeDtypeStruct((NC, NPAD, H), jnp.float32),
    mesh=_sc_mesh,
    compiler_params=_sc_params,
    scratch_types=[
        pltpu.VMEM((EPW,), jnp.int32),
        pltpu.VMEM((EPW,), jnp.float32),
        pltpu.VMEM((2, CHUNK, H), jnp.float32),
        pltpu.VMEM((CHUNK,), jnp.int32),
        pltpu.VMEM((CHUNK,), jnp.int32),
        pltpu.VMEM_SHARED((NPAD, H), jnp.float32),
        pltpu.SemaphoreType.DMA,
        pltpu.SemaphoreType.DMA,
        pltpu.SemaphoreType.DMA,
        pltpu.SemaphoreType.DMA,
        pltpu.SemaphoreType.DMA,
        pltpu.SemaphoreType.DMA,
    ],
)
def _scatter_msg(h_hbm, attn_hbm, src_hbm, dst_hbm, out_hbm,
                 src_v, attn_v, rows_v, didx0_v, didx1_v, msg_sp,
                 sem_i0, sem_i1, sem_g0, sem_g1, sem_s0, sem_s1):
    sem_i = (sem_i0, sem_i1)
    sem_g = (sem_g0, sem_g1)
    sem_s = (sem_s0, sem_s1)
    didx_v = (didx0_v, didx1_v)
    cid = lax.axis_index("c")
    sid = lax.axis_index("s")
    wid = sid * NC + cid

    def zbody(t, carry):
        r = t // (H // 16)
        c = t % (H // 16)
        rows_v[0, r, pl.ds(c * 16, 16)] = jnp.zeros((16,), jnp.float32)
        return carry

    lax.fori_loop(0, CHUNK * (H // 16), zbody, 0)
    rbase = sid * RPW
    for t in range(RPW // CHUNK):
        pltpu.sync_copy(rows_v.at[0], msg_sp.at[pl.ds(rbase + t * CHUNK, CHUNK)])

    ebase = wid * EPW
    pltpu.sync_copy(src_hbm.at[pl.ds(ebase, EPW)], src_v)
    pltpu.sync_copy(attn_hbm.at[pl.ds(ebase, EPW)], attn_v)
    plsc.subcore_barrier()

    def didx_copy(ci, b):
        return pltpu.make_async_copy(
            dst_hbm.at[pl.ds(ebase + ci * CHUNK, CHUNK)], didx_v[b],
            sem_i[b])

    def gather_copy(ci, b):
        return pltpu.make_async_copy(
            h_hbm.at[src_v.at[pl.ds(ci * CHUNK, CHUNK)]], rows_v.at[b],
            sem_g[b])

    def issue_chunk(ci, b):
        didx_copy(ci, b).start()
        gather_copy(ci, b).start()

    def scatter_copy(b):
        return pltpu.make_async_copy(rows_v.at[b], msg_sp.at[didx_v[b]],
                                     sem_s[b])

    def consume_chunk(ci, b):
        off = ci * CHUNK
        didx_copy(ci, b).wait()
        gather_copy(ci, b).wait()

        def scale_body(gg, c2):
            a16 = attn_v[pl.ds(off + gg * 16, 16)]
            for lane in range(16):
                a = a16[lane]
                k = gg * 16 + lane
                for j in range(H // 16):
                    sl = pl.ds(j * 16, 16)
                    rows_v[b, k, sl] = rows_v[b, k, sl] * a
            return c2

        lax.fori_loop(0, CHUNK // 16, scale_body, 0)
        scatter_copy(b).start(add=True)

    issue_chunk(0, 0)
    issue_chunk(1, 1)

    def outer(p, carry):
        ci0 = 2 * p
        consume_chunk(ci0, 0)
        consume_chunk(ci0 + 1, 1)
        scatter_copy(0).wait()
        issue_chunk(ci0 + 2, 0)
        scatter_copy(1).wait()

        @pl.when(p < NPAIR - 1)
        def _():
            issue_chunk(ci0 + 3, 1)

        return carry

    lax.fori_loop(0, NPAIR, outer, 0)
    consume_chunk(NCHUNK - 1, 0)
    scatter_copy(0).wait()
    plsc.subcore_barrier()
    pltpu.sync_copy(msg_sp.at[pl.ds(rbase, RPW)],
                    out_hbm.at[cid, pl.ds(rbase, RPW)])


def _proj_body(x_ref, wp_ref, bp_ref, wab_ref, bab_ref, h_ref, ab_ref):
    h = jnp.dot(x_ref[...], wp_ref[...], preferred_element_type=jnp.float32)
    h = h + bp_ref[...]
    h_ref[...] = h
    ab_ref[...] = jnp.dot(h, wab_ref[...],
                          preferred_element_type=jnp.float32) + bab_ref[...]


def _proj(x, wpT, bp, wab, bab):
    return pl.pallas_call(
        _proj_body,
        grid=(N // ROWBLK,),
        in_specs=[
            pl.BlockSpec((ROWBLK, H), lambda i: (i, 0)),
            pl.BlockSpec((H, H), lambda i: (0, 0)),
            pl.BlockSpec((1, H), lambda i: (0, 0)),
            pl.BlockSpec((H, 2), lambda i: (0, 0)),
            pl.BlockSpec((1, 2), lambda i: (0, 0)),
        ],
        out_specs=[
            pl.BlockSpec((ROWBLK, H), lambda i: (i, 0)),
            pl.BlockSpec((ROWBLK, 2), lambda i: (i, 0)),
        ],
        out_shape=[
            jax.ShapeDtypeStruct((N, H), jnp.float32),
            jax.ShapeDtypeStruct((N, 2), jnp.float32),
        ],
    )(x, wpT, bp, wab, bab)


def _softmax_body(z_ref, out_ref):
    z = z_ref[...]
    m = jnp.max(z)
    p = jnp.exp(z - m)
    out_ref[...] = p / jnp.sum(p)


def _softmax(z2d):
    return pl.pallas_call(
        _softmax_body,
        out_shape=jax.ShapeDtypeStruct(z2d.shape, jnp.float32),
    )(z2d)


def _update_body(h_ref, msg_ref, wc_ref, bc_ref, wab_ref, bab_ref,
                 h_out, ab_out):
    m = msg_ref[0] + msg_ref[1]
    u = jnp.dot(m, wc_ref[...], preferred_element_type=jnp.float32)
    u = u + bc_ref[...]
    h = h_ref[...] + jnp.maximum(u, 0.0)
    h_out[...] = h
    ab_out[...] = jnp.dot(h, wab_ref[...],
                          preferred_element_type=jnp.float32) + bab_ref[...]


def _update(h, msg2, wcT, bc, wab, bab):
    return pl.pallas_call(
        _update_body,
        grid=(N // ROWBLK,),
        in_specs=[
            pl.BlockSpec((ROWBLK, H), lambda i: (i, 0)),
            pl.BlockSpec((NC, ROWBLK, H), lambda i: (0, i, 0)),
            pl.BlockSpec((H, H), lambda i: (0, 0)),
            pl.BlockSpec((1, H), lambda i: (0, 0)),
            pl.BlockSpec((H, 2), lambda i: (0, 0)),
            pl.BlockSpec((1, 2), lambda i: (0, 0)),
        ],
        out_specs=[
            pl.BlockSpec((ROWBLK, H), lambda i: (i, 0)),
            pl.BlockSpec((ROWBLK, 2), lambda i: (i, 0)),
        ],
        out_shape=[
            jax.ShapeDtypeStruct((N, H), jnp.float32),
            jax.ShapeDtypeStruct((N, 2), jnp.float32),
        ],
    )(h, msg2, wcT, bc, wab, bab)


@jax.jit
def kernel(node_features, edge_index, W_proj, b_proj, W_convs, b_convs,
           W_atts, b_atts):
    src = edge_index[0]
    dst = edge_index[1]

    def wab_for(i):
        w1 = W_atts[i, 0, :H]
        w2 = W_atts[i, 0, H:]
        wab = jnp.stack([w1, w2], axis=1)
        bab = jnp.stack([b_atts[i, 0], jnp.float32(0.0)])
        return wab, bab.reshape(1, 2)

    wab0, bab0 = wab_for(0)
    h, ab = _proj(node_features, W_proj.T, b_proj.reshape(1, H), wab0, bab0)

    for i in range(L):
        z = _edge_logits(ab[:, 0], ab[:, 1], src, dst)
        attn = _softmax(z.reshape(E // H, H)).reshape(E)
        msg2 = _scatter_msg(h, attn, src, dst)
        if i + 1 < L:
            wab, bab = wab_for(i + 1)
        h, ab = _update(h, msg2, W_convs[i].T, b_convs[i].reshape(1, H),
                        wab, bab)
    return h


# speedup vs baseline: 1.0903x; 1.0903x over previous
import functools

import jax
import jax.numpy as jnp
from jax import lax
from jax.experimental import pallas as pl
from jax.experimental.pallas import tpu as pltpu
from jax.experimental.pallas import tpu_sc as plsc

N = 10000
E = 320000
H = 128
L = 3

NC = 2
NS = 16
NW = NC * NS
EPW = E // NW
CHUNK = 80
NCHUNK = EPW // CHUNK
NPAIR = NCHUNK // 2
NPAD = 10240
RPW = NPAD // NS
ZROWS = 128

ROWBLK = 2000

_sc_mesh = plsc.VectorSubcoreMesh(core_axis_name="c", subcore_axis_name="s")
_sc_params = pltpu.CompilerParams(needs_layout_passes=False)


@functools.partial(
    pl.kernel,
    out_type=jax.ShapeDtypeStruct((E,), jnp.float32),
    mesh=_sc_mesh,
    compiler_params=_sc_params,
    scratch_types=[
        pltpu.VMEM((N,), jnp.float32),
        pltpu.VMEM((N,), jnp.float32),
        pltpu.VMEM((EPW,), jnp.int32),
        pltpu.VMEM((EPW,), jnp.int32),
        pltpu.VMEM((EPW,), jnp.float32),
    ],
)
def _edge_logits(a_hbm, b_hbm, src_hbm, dst_hbm, z_hbm,
                 a_v, b_v, src_v, dst_v, z_v):
    wid = lax.axis_index("s") * NC + lax.axis_index("c")
    base = wid * EPW
    pltpu.sync_copy(a_hbm, a_v)
    pltpu.sync_copy(b_hbm, b_v)
    pltpu.sync_copy(src_hbm.at[pl.ds(base, EPW)], src_v)
    pltpu.sync_copy(dst_hbm.at[pl.ds(base, EPW)], dst_v)

    def body(k, carry):
        off = k * 16
        si = src_v[pl.ds(off, 16)]
        di = dst_v[pl.ds(off, 16)]
        a = plsc.load_gather(a_v, [si])
        b = plsc.load_gather(b_v, [di])
        z = a + b
        z_v[pl.ds(off, 16)] = jnp.where(z >= 0.0, z, 0.2 * z)
        return carry

    lax.fori_loop(0, EPW // 16, body, 0)
    pltpu.sync_copy(z_v, z_hbm.at[pl.ds(base, EPW)])


@functools.partial(
    pl.kernel,
    out_type=jax.ShapeDtypeStruct((NC, NPAD, H), jnp.float32),
    mesh=_sc_mesh,
    compiler_params=_sc_params,
    scratch_types=[
        pltpu.VMEM((EPW,), jnp.int32),
        pltpu.VMEM((EPW,), jnp.float32),
        pltpu.VMEM((2, CHUNK, H), jnp.float32),
        pltpu.VMEM((CHUNK,), jnp.int32),
        pltpu.VMEM((CHUNK,), jnp.int32),
        pltpu.VMEM_SHARED((NPAD, H), jnp.float32),
        pltpu.SemaphoreType.DMA,
        pltpu.SemaphoreType.DMA,
        pltpu.SemaphoreType.DMA,
        pltpu.SemaphoreType.DMA,
    ],
)
def _scatter_msg(h_hbm, attn_hbm, src_hbm, dst_hbm, out_hbm,
                 src_v, attn_v, rows_v, didx0_v, didx1_v, msg_sp,
                 sem_i0, sem_i1, sem_g0, sem_g1):
    sem_i = (sem_i0, sem_i1)
    sem_g = (sem_g0, sem_g1)
    didx_v = (didx0_v, didx1_v)
    cid = lax.axis_index("c")
    sid = lax.axis_index("s")
    wid = sid * NC + cid

    def zbody(t, carry):
        r = t // (H // 16)
        c = t % (H // 16)
        rows_v[0, r, pl.ds(c * 16, 16)] = jnp.zeros((16,), jnp.float32)
        return carry

    lax.fori_loop(0, CHUNK * (H // 16), zbody, 0)
    rbase = sid * RPW
    for t in range(RPW // CHUNK):
        pltpu.sync_copy(rows_v.at[0], msg_sp.at[pl.ds(rbase + t * CHUNK, CHUNK)])

    ebase = wid * EPW
    pltpu.sync_copy(src_hbm.at[pl.ds(ebase, EPW)], src_v)
    pltpu.sync_copy(attn_hbm.at[pl.ds(ebase, EPW)], attn_v)
    plsc.subcore_barrier()

    def didx_copy(ci, b):
        return pltpu.make_async_copy(
            dst_hbm.at[pl.ds(ebase + ci * CHUNK, CHUNK)], didx_v[b],
            sem_i[b])

    def gather_copy(ci, b):
        return pltpu.make_async_copy(
            h_hbm.at[src_v.at[pl.ds(ci * CHUNK, CHUNK)]], rows_v.at[b],
            sem_g[b])

    def issue_chunk(ci, b):
        didx_copy(ci, b).start()
        gather_copy(ci, b).start()

    def consume_chunk(ci, b):
        off = ci * CHUNK
        didx_copy(ci, b).wait()
        gather_copy(ci, b).wait()

        def scale_body(gg, c2):
            a16 = attn_v[pl.ds(off + gg * 16, 16)]
            for lane in range(16):
                a = a16[lane]
                k = gg * 16 + lane
                for j in range(H // 16):
                    sl = pl.ds(j * 16, 16)
                    rows_v[b, k, sl] = rows_v[b, k, sl] * a
            return c2

        lax.fori_loop(0, CHUNK // 16, scale_body, 0)
        pltpu.sync_copy(rows_v.at[b], msg_sp.at[didx_v[b]], add=True)

    issue_chunk(0, 0)

    def outer(p, carry):
        ci0 = 2 * p
        issue_chunk(ci0 + 1, 1)
        consume_chunk(ci0, 0)
        issue_chunk(ci0 + 2, 0)
        consume_chunk(ci0 + 1, 1)
        return carry

    lax.fori_loop(0, NPAIR, outer, 0)
    consume_chunk(NCHUNK - 1, 0)
    plsc.subcore_barrier()
    pltpu.sync_copy(msg_sp.at[pl.ds(rbase, RPW)],
                    out_hbm.at[cid, pl.ds(rbase, RPW)])


def _proj_body(x_ref, wp_ref, bp_ref, wab_ref, bab_ref, h_ref, ab_ref):
    h = jnp.dot(x_ref[...], wp_ref[...], preferred_element_type=jnp.float32)
    h = h + bp_ref[...]
    h_ref[...] = h
    ab_ref[...] = jnp.dot(h, wab_ref[...],
                          preferred_element_type=jnp.float32) + bab_ref[...]


def _proj(x, wpT, bp, wab, bab):
    return pl.pallas_call(
        _proj_body,
        grid=(N // ROWBLK,),
        in_specs=[
            pl.BlockSpec((ROWBLK, H), lambda i: (i, 0)),
            pl.BlockSpec((H, H), lambda i: (0, 0)),
            pl.BlockSpec((1, H), lambda i: (0, 0)),
            pl.BlockSpec((H, 2), lambda i: (0, 0)),
            pl.BlockSpec((1, 2), lambda i: (0, 0)),
        ],
        out_specs=[
            pl.BlockSpec((ROWBLK, H), lambda i: (i, 0)),
            pl.BlockSpec((ROWBLK, 2), lambda i: (i, 0)),
        ],
        out_shape=[
            jax.ShapeDtypeStruct((N, H), jnp.float32),
            jax.ShapeDtypeStruct((N, 2), jnp.float32),
        ],
    )(x, wpT, bp, wab, bab)


def _softmax_body(z_ref, out_ref):
    z = z_ref[...]
    m = jnp.max(z)
    p = jnp.exp(z - m)
    out_ref[...] = p / jnp.sum(p)


def _softmax(z2d):
    return pl.pallas_call(
        _softmax_body,
        out_shape=jax.ShapeDtypeStruct(z2d.shape, jnp.float32),
    )(z2d)


def _update_body(h_ref, msg_ref, wc_ref, bc_ref, wab_ref, bab_ref,
                 h_out, ab_out):
    m = msg_ref[0] + msg_ref[1]
    u = jnp.dot(m, wc_ref[...], preferred_element_type=jnp.float32)
    u = u + bc_ref[...]
    h = h_ref[...] + jnp.maximum(u, 0.0)
    h_out[...] = h
    ab_out[...] = jnp.dot(h, wab_ref[...],
                          preferred_element_type=jnp.float32) + bab_ref[...]


def _update(h, msg2, wcT, bc, wab, bab):
    return pl.pallas_call(
        _update_body,
        grid=(N // ROWBLK,),
        in_specs=[
            pl.BlockSpec((ROWBLK, H), lambda i: (i, 0)),
            pl.BlockSpec((NC, ROWBLK, H), lambda i: (0, i, 0)),
            pl.BlockSpec((H, H), lambda i: (0, 0)),
            pl.BlockSpec((1, H), lambda i: (0, 0)),
            pl.BlockSpec((H, 2), lambda i: (0, 0)),
            pl.BlockSpec((1, 2), lambda i: (0, 0)),
        ],
        out_specs=[
            pl.BlockSpec((ROWBLK, H), lambda i: (i, 0)),
            pl.BlockSpec((ROWBLK, 2), lambda i: (i, 0)),
        ],
        out_shape=[
            jax.ShapeDtypeStruct((N, H), jnp.float32),
            jax.ShapeDtypeStruct((N, 2), jnp.float32),
        ],
    )(h, msg2, wcT, bc, wab, bab)


@jax.jit
def kernel(node_features, edge_index, W_proj, b_proj, W_convs, b_convs,
           W_atts, b_atts):
    src = edge_index[0]
    dst = edge_index[1]

    def wab_for(i):
        w1 = W_atts[i, 0, :H]
        w2 = W_atts[i, 0, H:]
        wab = jnp.stack([w1, w2], axis=1)
        bab = jnp.stack([b_atts[i, 0], jnp.float32(0.0)])
        return wab, bab.reshape(1, 2)

    wab0, bab0 = wab_for(0)
    h, ab = _proj(node_features, W_proj.T, b_proj.reshape(1, H), wab0, bab0)

    for i in range(L):
        z = _edge_logits(ab[:, 0], ab[:, 1], src, dst)
        attn = _softmax(z.reshape(E // H, H)).reshape(E)
        msg2 = _scatter_msg(h, attn, src, dst)
        if i + 1 < L:
            wab, bab = wab_for(i + 1)
        h, ab = _update(h, msg2, W_convs[i].T, b_convs[i].reshape(1, H),
                        wab, bab)
    return h


# speedup vs baseline: 1.1909x; 1.0922x over previous
import functools

import jax
import jax.numpy as jnp
from jax import lax
from jax.experimental import pallas as pl
from jax.experimental.pallas import tpu as pltpu
from jax.experimental.pallas import tpu_sc as plsc

N = 10000
E = 320000
H = 128
L = 3

NC = 2
NS = 16
NW = NC * NS
EPW = E // NW
CHUNK = 80
NCHUNK = EPW // CHUNK
NPAIR = NCHUNK // 2
NPAD = 10240
RPW = NPAD // NS
ZROWS = 128

ROWBLK = 2000

_sc_mesh = plsc.VectorSubcoreMesh(core_axis_name="c", subcore_axis_name="s")
_sc_params = pltpu.CompilerParams(needs_layout_passes=False)


@functools.partial(
    pl.kernel,
    out_type=jax.ShapeDtypeStruct((E,), jnp.float32),
    mesh=_sc_mesh,
    compiler_params=_sc_params,
    scratch_types=[
        pltpu.VMEM((N,), jnp.float32),
        pltpu.VMEM((N,), jnp.float32),
        pltpu.VMEM((EPW,), jnp.int32),
        pltpu.VMEM((EPW,), jnp.int32),
        pltpu.VMEM((EPW,), jnp.float32),
    ],
)
def _edge_logits(a_hbm, b_hbm, src_hbm, dst_hbm, z_hbm,
                 a_v, b_v, src_v, dst_v, z_v):
    wid = lax.axis_index("s") * NC + lax.axis_index("c")
    base = wid * EPW
    pltpu.sync_copy(a_hbm, a_v)
    pltpu.sync_copy(b_hbm, b_v)
    pltpu.sync_copy(src_hbm.at[pl.ds(base, EPW)], src_v)
    pltpu.sync_copy(dst_hbm.at[pl.ds(base, EPW)], dst_v)

    def body(k, carry):
        off = k * 16
        si = src_v[pl.ds(off, 16)]
        di = dst_v[pl.ds(off, 16)]
        a = plsc.load_gather(a_v, [si])
        b = plsc.load_gather(b_v, [di])
        z = a + b
        z_v[pl.ds(off, 16)] = jnp.where(z >= 0.0, z, 0.2 * z)
        return carry

    lax.fori_loop(0, EPW // 16, body, 0)
    pltpu.sync_copy(z_v, z_hbm.at[pl.ds(base, EPW)])


@functools.partial(
    pl.kernel,
    out_type=jax.ShapeDtypeStruct((NC, NPAD, H), jnp.float32),
    mesh=_sc_mesh,
    compiler_params=_sc_params,
    scratch_types=[
        pltpu.VMEM((EPW,), jnp.int32),
        pltpu.VMEM((3, CHUNK, H), jnp.float32),
        pltpu.VMEM((CHUNK,), jnp.int32),
        pltpu.VMEM((CHUNK,), jnp.int32),
        pltpu.VMEM((CHUNK,), jnp.int32),
        pltpu.VMEM((CHUNK,), jnp.float32),
        pltpu.VMEM((CHUNK,), jnp.float32),
        pltpu.VMEM((CHUNK,), jnp.float32),
        pltpu.VMEM_SHARED((NPAD, H), jnp.float32),
    ] + [pltpu.SemaphoreType.DMA] * 12,
)
def _scatter_msg(h_hbm, attn_hbm, src_hbm, dst_hbm, out_hbm,
                 src_v, rows_v, di0, di1, di2, at0, at1, at2, msg_sp, *sems):
    sem_d = sems[0:3]
    sem_a = sems[3:6]
    sem_g = sems[6:9]
    sem_s = sems[9:12]
    didx_v = (di0, di1, di2)
    attn_c = (at0, at1, at2)
    cid = lax.axis_index("c")
    sid = lax.axis_index("s")
    wid = sid * NC + cid

    def zbody(t, carry):
        r = t // (H // 16)
        c = t % (H // 16)
        rows_v[0, r, pl.ds(c * 16, 16)] = jnp.zeros((16,), jnp.float32)
        return carry

    lax.fori_loop(0, CHUNK * (H // 16), zbody, 0)
    rbase = sid * RPW
    for t in range(RPW // CHUNK):
        pltpu.sync_copy(rows_v.at[0], msg_sp.at[pl.ds(rbase + t * CHUNK, CHUNK)])

    ebase = wid * EPW
    pltpu.sync_copy(src_hbm.at[pl.ds(ebase, EPW)], src_v)
    plsc.subcore_barrier()

    def didx_copy(ci, b):
        return pltpu.make_async_copy(
            dst_hbm.at[pl.ds(ebase + ci * CHUNK, CHUNK)], didx_v[b],
            sem_d[b])

    def attn_copy(ci, b):
        return pltpu.make_async_copy(
            attn_hbm.at[pl.ds(ebase + ci * CHUNK, CHUNK)], attn_c[b],
            sem_a[b])

    def gather_copy(ci, b):
        return pltpu.make_async_copy(
            h_hbm.at[src_v.at[pl.ds(ci * CHUNK, CHUNK)]], rows_v.at[b],
            sem_g[b])

    def scat_copy(b):
        return pltpu.make_async_copy(rows_v.at[b], msg_sp.at[didx_v[b]],
                                     sem_s[b])

    def issue_chunk(ci, b):
        didx_copy(ci, b).start()
        attn_copy(ci, b).start()
        gather_copy(ci, b).start()

    def process(ci, b):
        didx_copy(ci, b).wait()
        attn_copy(ci, b).wait()
        gather_copy(ci, b).wait()

        def scale_body(gg, c2):
            a16 = attn_c[b][pl.ds(gg * 16, 16)]
            for lane in range(16):
                a = a16[lane]
                k = gg * 16 + lane
                for j in range(H // 16):
                    sl = pl.ds(j * 16, 16)
                    rows_v[b, k, sl] = rows_v[b, k, sl] * a
            return c2

        lax.fori_loop(0, CHUNK // 16, scale_body, 0)
        scat_copy(b).start(add=True)

    for b in range(3):
        issue_chunk(b, b)
    process(0, 0)
    process(1, 1)
    scat_copy(0).wait()
    issue_chunk(3, 0)
    process(2, 2)
    scat_copy(1).wait()
    issue_chunk(4, 1)

    def outer(g, carry):
        c0 = 3 * g
        for b in range(3):
            process(c0 + b, b)
            bp = (b + 2) % 3
            scat_copy(bp).wait()
            issue_chunk(c0 + b + 2, bp)
        return carry

    lax.fori_loop(1, (NCHUNK - 2) // 3, outer, 0)

    process(NCHUNK - 2, 0)
    scat_copy(2).wait()
    process(NCHUNK - 1, 1)
    scat_copy(0).wait()
    scat_copy(1).wait()
    plsc.subcore_barrier()
    pltpu.sync_copy(msg_sp.at[pl.ds(rbase, RPW)],
                    out_hbm.at[cid, pl.ds(rbase, RPW)])


def _proj_body(x_ref, wp_ref, bp_ref, wab_ref, bab_ref, h_ref, ab_ref):
    h = jnp.dot(x_ref[...], wp_ref[...], preferred_element_type=jnp.float32)
    h = h + bp_ref[...]
    h_ref[...] = h
    ab_ref[...] = jnp.dot(h, wab_ref[...],
                          preferred_element_type=jnp.float32) + bab_ref[...]


def _proj(x, wpT, bp, wab, bab):
    return pl.pallas_call(
        _proj_body,
        grid=(N // ROWBLK,),
        in_specs=[
            pl.BlockSpec((ROWBLK, H), lambda i: (i, 0)),
            pl.BlockSpec((H, H), lambda i: (0, 0)),
            pl.BlockSpec((1, H), lambda i: (0, 0)),
            pl.BlockSpec((H, 2), lambda i: (0, 0)),
            pl.BlockSpec((1, 2), lambda i: (0, 0)),
        ],
        out_specs=[
            pl.BlockSpec((ROWBLK, H), lambda i: (i, 0)),
            pl.BlockSpec((ROWBLK, 2), lambda i: (i, 0)),
        ],
        out_shape=[
            jax.ShapeDtypeStruct((N, H), jnp.float32),
            jax.ShapeDtypeStruct((N, 2), jnp.float32),
        ],
    )(x, wpT, bp, wab, bab)


def _softmax_body(z_ref, out_ref):
    z = z_ref[...]
    m = jnp.max(z)
    p = jnp.exp(z - m)
    out_ref[...] = p / jnp.sum(p)


def _softmax(z2d):
    return pl.pallas_call(
        _softmax_body,
        out_shape=jax.ShapeDtypeStruct(z2d.shape, jnp.float32),
    )(z2d)


def _update_body(h_ref, msg_ref, wc_ref, bc_ref, wab_ref, bab_ref,
                 h_out, ab_out):
    m = msg_ref[0] + msg_ref[1]
    u = jnp.dot(m, wc_ref[...], preferred_element_type=jnp.float32)
    u = u + bc_ref[...]
    h = h_ref[...] + jnp.maximum(u, 0.0)
    h_out[...] = h
    ab_out[...] = jnp.dot(h, wab_ref[...],
                          preferred_element_type=jnp.float32) + bab_ref[...]


def _update(h, msg2, wcT, bc, wab, bab):
    return pl.pallas_call(
        _update_body,
        grid=(N // ROWBLK,),
        in_specs=[
            pl.BlockSpec((ROWBLK, H), lambda i: (i, 0)),
            pl.BlockSpec((NC, ROWBLK, H), lambda i: (0, i, 0)),
            pl.BlockSpec((H, H), lambda i: (0, 0)),
            pl.BlockSpec((1, H), lambda i: (0, 0)),
            pl.BlockSpec((H, 2), lambda i: (0, 0)),
            pl.BlockSpec((1, 2), lambda i: (0, 0)),
        ],
        out_specs=[
            pl.BlockSpec((ROWBLK, H), lambda i: (i, 0)),
            pl.BlockSpec((ROWBLK, 2), lambda i: (i, 0)),
        ],
        out_shape=[
            jax.ShapeDtypeStruct((N, H), jnp.float32),
            jax.ShapeDtypeStruct((N, 2), jnp.float32),
        ],
    )(h, msg2, wcT, bc, wab, bab)


@jax.jit
def kernel(node_features, edge_index, W_proj, b_proj, W_convs, b_convs,
           W_atts, b_atts):
    src = edge_index[0]
    dst = edge_index[1]

    def wab_for(i):
        w1 = W_atts[i, 0, :H]
        w2 = W_atts[i, 0, H:]
        wab = jnp.stack([w1, w2], axis=1)
        bab = jnp.stack([b_atts[i, 0], jnp.float32(0.0)])
        return wab, bab.reshape(1, 2)

    wab0, bab0 = wab_for(0)
    h, ab = _proj(node_features, W_proj.T, b_proj.reshape(1, H), wab0, bab0)

    for i in range(L):
        z = _edge_logits(ab[:, 0], ab[:, 1], src, dst)
        attn = _softmax(z.reshape(E // H, H)).reshape(E)
        msg2 = _scatter_msg(h, attn, src, dst)
        if i + 1 < L:
            wab, bab = wab_for(i + 1)
        h, ab = _update(h, msg2, W_convs[i].T, b_convs[i].reshape(1, H),
                        wab, bab)
    return h
